# baseline (speedup 1.0000x reference)
import functools

import jax
import jax.numpy as jnp
from jax import lax
from jax.experimental import pallas as pl
from jax.experimental.pallas import tpu as pltpu
from jax.experimental.pallas import tpu_sc as plsc

NUM_CORES = 2
NUM_SUBCORES = 16
NW = NUM_CORES * NUM_SUBCORES
LANES = 16
F = 64
BW = 512
BATCH = 16384
N_USERS = 1000000
N_ITEMS = 100000
XBLOCKS = (N_USERS // BW // NW) * NW
YBLOCKS = (N_ITEMS // BW // NW) * NW
X_TAIL0 = XBLOCKS * BW
Y_TAIL0 = YBLOCKS * BW
WSZ = 1568
TWSZ = 528
PAD_IDX = 2048
RING = 8


def _iota():
    return lax.broadcasted_iota(jnp.int32, (LANES,), 0)


def _rot_factory(iota):
    rot_idx = [((iota + k) & (LANES - 1)).reshape(LANES, 1)
               for k in (8, 4, 2, 1)]
    gdn = lax.GatherDimensionNumbers(offset_dims=(), collapsed_slice_dims=(0,),
                                     start_index_map=(0,))

    def rot_reduce(p):
        for ridx in rot_idx:
            p = p + lax.gather(p, ridx, gdn, slice_sizes=(1,),
                               mode=lax.GatherScatterMode.PROMISE_IN_BOUNDS)
        return p
    return rot_reduce


def _gather_body(su_hbm, ord_hbm, si_hbm, iord_hbm, sx_hbm, sy_hbm,
                 xt_hbm, yt_hbm, xtail_hbm, ytail_hbm,
                 ue_hbm, ie_hbm,
                 bufs, sxv, syv, suw, ordw, rowring, sem0, sem1, semrow):
    wid = lax.axis_index("s") * NUM_CORES + lax.axis_index("c")
    iota = _iota()
    sems = (sem0, sem1)

    pltpu.sync_copy(sx_hbm, sxv)
    pltpu.sync_copy(sy_hbm, syv)

    def phase(nb_per_tile, blk0, table_hbm, starts_v, skey_hbm, sord_hbm,
              out_hbm, tail0, tail_hbm):
        k_begin = starts_v[pl.ds(blk0, LANES)][0]
        k0a = (k_begin // 8) * 8
        pltpu.sync_copy(skey_hbm.at[pl.ds(k0a, WSZ)], suw)
        pltpu.sync_copy(sord_hbm.at[pl.ds(k0a, WSZ)], ordw)

        def extract(kb, s):
            blk = blk0 + kb
            sv = starts_v[pl.ds(blk, LANES)]
            s0, s1 = sv[0], sv[1]

            def el(k, _):
                u = suw[pl.ds(k - k0a, LANES)][0]
                b = ordw[pl.ds(k - k0a, LANES)][0]
                c = jnp.broadcast_to(u - blk * BW, (LANES,))
                slot = k & (RING - 1)
                for t in range(F // LANES):
                    v = plsc.load_gather(bufs.at[s], [iota + t * LANES, c])
                    rowring[slot, pl.ds(t * LANES, LANES)] = v
                pltpu.async_copy(rowring.at[slot],
                                 out_hbm.at[pl.ds(b * F, F)], semrow)

                @pl.when(k - s0 >= RING)
                def _():
                    pltpu.make_async_copy(
                        tail_hbm.at[pl.ds(0, F)], rowring.at[0], semrow).wait()
                return 0

            lax.fori_loop(s0, s1, el, 0)

            def dr(_, __):
                pltpu.make_async_copy(
                    tail_hbm.at[pl.ds(0, F)], rowring.at[0], semrow).wait()
                return 0
            lax.fori_loop(0, jnp.minimum(s1 - s0, RING), dr, 0)

        def issue(kb, s):
            off = pl.multiple_of((blk0 + kb) * BW, 128)
            pltpu.async_copy(table_hbm.at[:, pl.ds(off, BW)], bufs.at[s],
                             sems[s])

        def drain_block(s):
            pltpu.make_async_copy(table_hbm.at[:, pl.ds(0, BW)], bufs.at[s],
                                  sems[s]).wait()

        issue(0, 0)
        issue(1, 1)

        def body(it, _):
            kb0 = it * 2
            drain_block(0)
            extract(kb0, 0)

            @pl.when(kb0 + 2 < nb_per_tile)
            def _():
                issue(kb0 + 2, 0)
            kb1 = kb0 + 1

            @pl.when(kb1 < nb_per_tile)
            def _():
                drain_block(1)
                extract(kb1, 1)

                @pl.when(kb1 + 2 < nb_per_tile)
                def _():
                    issue(kb1 + 2, 1)
            return 0

        lax.fori_loop(0, (nb_per_tile + 1) // 2, body, 0)

        t0 = starts_v[pl.ds(nb_per_tile * NW, LANES)][0]
        cnt = BATCH - t0
        take = (cnt + NW - 1) // NW
        kt0 = t0 + wid * take
        kt1 = jnp.minimum(kt0 + take, BATCH)
        pltpu.sync_copy(skey_hbm.at[pl.ds(BATCH - TWSZ + LANES, TWSZ)],
                        suw.at[pl.ds(0, TWSZ)])
        pltpu.sync_copy(sord_hbm.at[pl.ds(BATCH - TWSZ + LANES, TWSZ)],
                        ordw.at[pl.ds(0, TWSZ)])

        def tel(k, _):
            w = k - (BATCH - TWSZ + LANES)
            u = suw[pl.ds(w, LANES)][0]
            b = ordw[pl.ds(w, LANES)][0]
            pltpu.sync_copy(tail_hbm.at[pl.ds((u - tail0) * F, F)],
                            rowring.at[0])
            pltpu.sync_copy(rowring.at[0], out_hbm.at[pl.ds(b * F, F)])
            return 0
        lax.fori_loop(kt0, kt1, tel, 0)

    phase(XBLOCKS // NW, wid * (XBLOCKS // NW), xt_hbm, sxv, su_hbm, ord_hbm,
          ue_hbm, X_TAIL0, xtail_hbm)
    phase(YBLOCKS // NW, wid * (YBLOCKS // NW), yt_hbm, syv, si_hbm, iord_hbm,
          ie_hbm, Y_TAIL0, ytail_hbm)


def _dot_body(ue_hbm, ie_hbm, out_hbm, uebuf, iebuf, outv):
    wid = lax.axis_index("s") * NUM_CORES + lax.axis_index("c")
    n_el = outv.shape[0]
    base = wid * n_el
    iota = _iota()
    rot_reduce = _rot_factory(iota)
    pltpu.sync_copy(ue_hbm.at[pl.ds(base * F, n_el * F)], uebuf)
    pltpu.sync_copy(ie_hbm.at[pl.ds(base * F, n_el * F)], iebuf)

    def group(g, _):
        acc = jnp.zeros((LANES,), jnp.float32)
        for l in range(LANES):
            r = g * LANES + l
            p = None
            for t in range(F // LANES):
                q = (uebuf[pl.ds(r * F + t * LANES, LANES)]
                     * iebuf[pl.ds(r * F + t * LANES, LANES)])
                p = q if p is None else p + q
            p = rot_reduce(p)
            acc = jnp.where(iota == l, p, acc)
        outv[pl.ds(g * LANES, LANES)] = acc
        return 0

    lax.fori_loop(0, n_el // LANES, group, 0)
    pltpu.sync_copy(outv, out_hbm.at[pl.ds(base, n_el)])


def _mesh():
    return plsc.VectorSubcoreMesh(core_axis_name="c", subcore_axis_name="s",
                                  num_cores=NUM_CORES,
                                  num_subcores=NUM_SUBCORES)


@functools.cache
def _build_gather():
    return pl.kernel(
        _gather_body,
        out_type=(jax.ShapeDtypeStruct((BATCH * F,), jnp.float32),
                  jax.ShapeDtypeStruct((BATCH * F,), jnp.float32)),
        mesh=_mesh(),
        scratch_types=[
            pltpu.VMEM((2, F, BW), jnp.float32),
            pltpu.VMEM((2000,), jnp.int32),
            pltpu.VMEM((224,), jnp.int32),
            pltpu.VMEM((WSZ,), jnp.int32),
            pltpu.VMEM((WSZ,), jnp.int32),
            pltpu.VMEM((RING, F), jnp.float32),
            pltpu.SemaphoreType.DMA,
            pltpu.SemaphoreType.DMA,
            pltpu.SemaphoreType.DMA,
        ],
        compiler_params=pltpu.CompilerParams(use_tc_tiling_on_sc=True,
                                             needs_layout_passes=False),
    )


@functools.cache
def _build_dot():
    n_el = BATCH // NW
    return pl.kernel(
        _dot_body,
        out_type=jax.ShapeDtypeStruct((BATCH,), jnp.float32),
        mesh=_mesh(),
        scratch_types=[
            pltpu.VMEM((n_el * F,), jnp.float32),
            pltpu.VMEM((n_el * F,), jnp.float32),
            pltpu.VMEM((n_el,), jnp.float32),
        ],
        compiler_params=pltpu.CompilerParams(use_tc_tiling_on_sc=True),
    )


def kernel(users, items, x, y):
    users = users.astype(jnp.int32)
    items = items.astype(jnp.int32)
    order = jnp.argsort(users >> 9)
    su = jnp.take(users, order)
    iorder = jnp.argsort(items >> 9)
    si = jnp.take(items, iorder)
    sx = jnp.searchsorted(su, jnp.arange(XBLOCKS + 1, dtype=jnp.int32) * BW
                          ).astype(jnp.int32)
    sy = jnp.searchsorted(si, jnp.arange(YBLOCKS + 1, dtype=jnp.int32) * BW
                          ).astype(jnp.int32)
    sx = jnp.concatenate([sx, jnp.full((2000 - XBLOCKS - 1,), BATCH,
                                       jnp.int32)])
    sy = jnp.concatenate([sy, jnp.full((224 - YBLOCKS - 1,), BATCH,
                                       jnp.int32)])
    zpad = jnp.zeros((PAD_IDX,), jnp.int32)
    su_p = jnp.concatenate([su, zpad])
    ord_p = jnp.concatenate([order.astype(jnp.int32), zpad])
    si_p = jnp.concatenate([si, zpad])
    iord_p = jnp.concatenate([iorder.astype(jnp.int32), zpad])
    xtail = lax.slice(x, (X_TAIL0, 0), (N_USERS, F)).reshape(-1)
    ytail = lax.slice(y, (Y_TAIL0, 0), (N_ITEMS, F)).reshape(-1)
    ue, ie = _build_gather()(su_p, ord_p, si_p, iord_p, sx, sy,
                             x.T, y.T, xtail, ytail)
    return _build_dot()(ue, ie)


# speedup vs baseline: 2.1737x; 2.1737x over previous
import functools

import jax
import jax.numpy as jnp
from jax import lax
from jax.experimental import pallas as pl
from jax.experimental.pallas import tpu as pltpu
from jax.experimental.pallas import tpu_sc as plsc

NUM_CORES = 2
NUM_SUBCORES = 16
NW = NUM_CORES * NUM_SUBCORES
LANES = 16
F = 64
BW = 512
BATCH = 16384
N_USERS = 1000000
N_ITEMS = 100000
XB_PER = N_USERS // BW // NW
X_TAIL0 = XB_PER * NW * BW
X_RANGE = XB_PER * BW
YB_TOT = N_ITEMS // BW
Y_TAIL0 = YB_TOT * BW
YB_ITERS = YB_TOT // NW + 1
LOC = 784
RING = 8


def _iota():
    return lax.broadcasted_iota(jnp.int32, (LANES,), 0)


def _rot_factory(iota):
    rot_idx = [((iota + k) & (LANES - 1)).reshape(LANES, 1)
               for k in (8, 4, 2, 1)]
    gdn = lax.GatherDimensionNumbers(offset_dims=(), collapsed_slice_dims=(0,),
                                     start_index_map=(0,))

    def rot_reduce(p):
        for ridx in rot_idx:
            p = p + lax.gather(p, ridx, gdn, slice_sizes=(1,),
                               mode=lax.GatherScatterMode.PROMISE_IN_BOUNDS)
        return p
    return rot_reduce


def _gather_body(users_hbm, items_hbm, xt_hbm, yt_hbm, xtail_hbm, ytail_hbm,
                 ue_hbm, ie_hbm,
                 bufs, uall, locu, locb, tmpu, tmpb, rowring,
                 sem0, sem1, semrow):
    wid = lax.axis_index("s") * NUM_CORES + lax.axis_index("c")
    iota = _iota()
    sems = (sem0, sem1)

    def phase(idx_hbm, select_mask, nb_iters, blk_of, blk_valid,
              table_hbm, out_hbm, tail0, tail_hbm, tail_wid):
        pltpu.sync_copy(idx_hbm, uall)

        def scan(i, off):
            uv = uall[pl.ds(i * LANES, LANES)]
            bv = iota + i * LANES
            m = select_mask(uv)
            plsc.store_compressed(locu.at[pl.ds(off, LANES)], uv, mask=m)
            plsc.store_compressed(locb.at[pl.ds(off, LANES)], bv, mask=m)
            return off + plsc.all_reduce_population_count(m)[0]

        nloc = lax.fori_loop(0, BATCH // LANES, scan, 0)
        nj = (nloc + LANES - 1) // LANES

        def extract(kb, s):
            blk = blk_of(kb)

            def jloop(j, cnt):
                uv = locu[pl.ds(j * LANES, LANES)]
                bv = locb[pl.ds(j * LANES, LANES)]
                m = ((uv >= blk * BW) & (uv < (blk + 1) * BW)
                     & (iota < nloc - j * LANES))
                plsc.store_compressed(tmpu.at[pl.ds(0, LANES)], uv, mask=m)
                plsc.store_compressed(tmpb.at[pl.ds(0, LANES)], bv, mask=m)
                n = plsc.all_reduce_population_count(m)[0]

                def el(e, cnt2):
                    u = tmpu[pl.ds(e, LANES)][0]
                    b = tmpb[pl.ds(e, LANES)][0]
                    c = jnp.broadcast_to(u - blk * BW, (LANES,))
                    slot = cnt2 & (RING - 1)
                    for t in range(F // LANES):
                        v = plsc.load_gather(bufs.at[s],
                                             [iota + t * LANES, c])
                        rowring[slot, pl.ds(t * LANES, LANES)] = v
                    pltpu.async_copy(rowring.at[slot],
                                     out_hbm.at[pl.ds(b * F, F)], semrow)

                    @pl.when(cnt2 >= RING)
                    def _():
                        pltpu.make_async_copy(tail_hbm.at[pl.ds(0, F)],
                                              rowring.at[0], semrow).wait()
                    return cnt2 + 1

                return lax.fori_loop(0, n, el, cnt)

            cnt_end = lax.fori_loop(0, nj, jloop, 0)

            def dr(_, __):
                pltpu.make_async_copy(tail_hbm.at[pl.ds(0, F)],
                                      rowring.at[0], semrow).wait()
                return 0
            lax.fori_loop(0, jnp.minimum(cnt_end, RING), dr, 0)

        def issue(kb, s):
            off = pl.multiple_of(blk_of(kb) * BW, 128)
            pltpu.async_copy(table_hbm.at[:, pl.ds(off, BW)], bufs.at[s],
                             sems[s])

        def drain_block(s):
            pltpu.make_async_copy(table_hbm.at[:, pl.ds(0, BW)], bufs.at[s],
                                  sems[s]).wait()

        issue(0, 0)
        issue(1, 1)

        def body(it, _):
            kb0 = it * 2

            @pl.when(blk_valid(kb0))
            def _():
                drain_block(0)
                extract(kb0, 0)

                @pl.when(blk_valid(kb0 + 2))
                def _():
                    issue(kb0 + 2, 0)
            kb1 = kb0 + 1

            @pl.when(blk_valid(kb1))
            def _():
                drain_block(1)
                extract(kb1, 1)

                @pl.when(blk_valid(kb1 + 2))
                def _():
                    issue(kb1 + 2, 1)
            return 0

        lax.fori_loop(0, (nb_iters + 1) // 2, body, 0)

        @pl.when(wid == tail_wid)
        def _():
            def tj(j, _):
                uv = locu[pl.ds(j * LANES, LANES)]
                bv = locb[pl.ds(j * LANES, LANES)]
                m = (uv >= tail0) & (iota < nloc - j * LANES)
                plsc.store_compressed(tmpu.at[pl.ds(0, LANES)], uv, mask=m)
                plsc.store_compressed(tmpb.at[pl.ds(0, LANES)], bv, mask=m)
                n = plsc.all_reduce_population_count(m)[0]

                def tel(e, __):
                    u = tmpu[pl.ds(e, LANES)][0]
                    b = tmpb[pl.ds(e, LANES)][0]
                    pltpu.sync_copy(tail_hbm.at[pl.ds((u - tail0) * F, F)],
                                    rowring.at[0])
                    pltpu.sync_copy(rowring.at[0],
                                    out_hbm.at[pl.ds(b * F, F)])
                    return 0
                lax.fori_loop(0, n, tel, 0)
                return 0
            lax.fori_loop(0, nj, tj, 0)

    xlo = wid * X_RANGE
    xhi = jnp.where(wid == NW - 1, N_USERS, xlo + X_RANGE)
    phase(users_hbm, lambda uv: (uv >= xlo) & (uv < xhi),
          XB_PER, lambda kb: wid * XB_PER + kb,
          lambda kb: kb < XB_PER,
          xt_hbm, ue_hbm, X_TAIL0, xtail_hbm, NW - 1)

    def ysel(uv):
        m = ((uv // BW) % NW == wid) & (uv < Y_TAIL0)
        return m | ((uv >= Y_TAIL0) & (wid == 3))

    phase(items_hbm, ysel,
          YB_ITERS, lambda kb: kb * NW + wid,
          lambda kb: (kb < YB_ITERS) & (kb * NW + wid < YB_TOT),
          yt_hbm, ie_hbm, Y_TAIL0, ytail_hbm, 3)


def _dot_body(ue_hbm, ie_hbm, out_hbm, uebuf, iebuf, outv):
    wid = lax.axis_index("s") * NUM_CORES + lax.axis_index("c")
    n_el = outv.shape[0]
    base = wid * n_el
    iota = _iota()
    rot_reduce = _rot_factory(iota)
    pltpu.sync_copy(ue_hbm.at[pl.ds(base * F, n_el * F)], uebuf)
    pltpu.sync_copy(ie_hbm.at[pl.ds(base * F, n_el * F)], iebuf)

    def group(g, _):
        acc = jnp.zeros((LANES,), jnp.float32)
        for l in range(LANES):
            r = g * LANES + l
            p = None
            for t in range(F // LANES):
                q = (uebuf[pl.ds(r * F + t * LANES, LANES)]
                     * iebuf[pl.ds(r * F + t * LANES, LANES)])
                p = q if p is None else p + q
            p = rot_reduce(p)
            acc = jnp.where(iota == l, p, acc)
        outv[pl.ds(g * LANES, LANES)] = acc
        return 0

    lax.fori_loop(0, n_el // LANES, group, 0)
    pltpu.sync_copy(outv, out_hbm.at[pl.ds(base, n_el)])


def _mesh():
    return plsc.VectorSubcoreMesh(core_axis_name="c", subcore_axis_name="s",
                                  num_cores=NUM_CORES,
                                  num_subcores=NUM_SUBCORES)


@functools.cache
def _build_gather():
    return pl.kernel(
        _gather_body,
        out_type=(jax.ShapeDtypeStruct((BATCH * F,), jnp.float32),
                  jax.ShapeDtypeStruct((BATCH * F,), jnp.float32)),
        mesh=_mesh(),
        scratch_types=[
            pltpu.VMEM((2, F, BW), jnp.float32),
            pltpu.VMEM((BATCH,), jnp.int32),
            pltpu.VMEM((LOC + LANES,), jnp.int32),
            pltpu.VMEM((LOC + LANES,), jnp.int32),
            pltpu.VMEM((2 * LANES,), jnp.int32),
            pltpu.VMEM((2 * LANES,), jnp.int32),
            pltpu.VMEM((RING, F), jnp.float32),
            pltpu.SemaphoreType.DMA,
            pltpu.SemaphoreType.DMA,
            pltpu.SemaphoreType.DMA,
        ],
        compiler_params=pltpu.CompilerParams(use_tc_tiling_on_sc=True,
                                             needs_layout_passes=False),
    )


@functools.cache
def _build_dot():
    n_el = BATCH // NW
    return pl.kernel(
        _dot_body,
        out_type=jax.ShapeDtypeStruct((BATCH,), jnp.float32),
        mesh=_mesh(),
        scratch_types=[
            pltpu.VMEM((n_el * F,), jnp.float32),
            pltpu.VMEM((n_el * F,), jnp.float32),
            pltpu.VMEM((n_el,), jnp.float32),
        ],
        compiler_params=pltpu.CompilerParams(use_tc_tiling_on_sc=True,
                                             needs_layout_passes=False),
    )


def kernel(users, items, x, y):
    users = users.astype(jnp.int32)
    items = items.astype(jnp.int32)
    xtail = lax.slice(x, (X_TAIL0, 0), (N_USERS, F)).reshape(-1)
    ytail = lax.slice(y, (Y_TAIL0, 0), (N_ITEMS, F)).reshape(-1)
    ue, ie = _build_gather()(users, items, x.T, y.T, xtail, ytail)
    return _build_dot()(ue, ie)


# speedup vs baseline: 2.3517x; 1.0819x over previous
import functools

import jax
import jax.numpy as jnp
from jax import lax
from jax.experimental import pallas as pl
from jax.experimental.pallas import tpu as pltpu
from jax.experimental.pallas import tpu_sc as plsc

NUM_CORES = 2
NUM_SUBCORES = 16
NW = NUM_CORES * NUM_SUBCORES
LANES = 16
F = 64
BW = 512
BATCH = 16384
N_USERS = 1000000
N_ITEMS = 100000
XB_PER = N_USERS // BW // NW
X_TAIL0 = XB_PER * NW * BW
X_RANGE = XB_PER * BW
YB_TOT = N_ITEMS // BW
Y_TAIL0 = YB_TOT * BW
YB_ITERS = YB_TOT // NW + 1
LOC = 784
RING = 8
DEPTH = 3


def _iota():
    return lax.broadcasted_iota(jnp.int32, (LANES,), 0)


def _rot_factory(iota):
    rot_idx = [((iota + k) & (LANES - 1)).reshape(LANES, 1)
               for k in (8, 4, 2, 1)]
    gdn = lax.GatherDimensionNumbers(offset_dims=(), collapsed_slice_dims=(0,),
                                     start_index_map=(0,))

    def rot_reduce(p):
        for ridx in rot_idx:
            p = p + lax.gather(p, ridx, gdn, slice_sizes=(1,),
                               mode=lax.GatherScatterMode.PROMISE_IN_BOUNDS)
        return p
    return rot_reduce


def _gather_body(users_hbm, items_hbm, xt_hbm, yt_hbm, xtail_hbm, ytail_hbm,
                 ue_hbm, ie_hbm,
                 bufs, uall, locu, locb, tmpu, tmpb, rowring,
                 sem0, sem1, sem2, semrow):
    wid = lax.axis_index("s") * NUM_CORES + lax.axis_index("c")
    iota = _iota()
    sems = (sem0, sem1, sem2)

    def phase(idx_hbm, select_mask, nb_iters, blk_of, blk_valid,
              table_hbm, out_hbm, tail0, tail_hbm, tail_wid):
        def issue(kb, s):
            off = pl.multiple_of(blk_of(kb) * BW, 128)
            pltpu.async_copy(table_hbm.at[:, pl.ds(off, BW)], bufs.at[s],
                             sems[s])

        def drain_block(s):
            pltpu.make_async_copy(table_hbm.at[:, pl.ds(0, BW)], bufs.at[s],
                                  sems[s]).wait()

        for s in range(DEPTH):
            issue(s, s)

        pltpu.sync_copy(idx_hbm, uall)

        def scan(i, off):
            uv = uall[pl.ds(i * LANES, LANES)]
            bv = iota + i * LANES
            m = select_mask(uv)
            plsc.store_compressed(locu.at[pl.ds(off, LANES)], uv, mask=m)
            plsc.store_compressed(locb.at[pl.ds(off, LANES)], bv, mask=m)
            return off + plsc.all_reduce_population_count(m)[0]

        nloc = lax.fori_loop(0, BATCH // LANES, scan, 0)
        nj = (nloc + LANES - 1) // LANES

        def extract(kb, s):
            blk = blk_of(kb)

            def jloop(j, cnt):
                uv = locu[pl.ds(j * LANES, LANES)]
                bv = locb[pl.ds(j * LANES, LANES)]
                m = ((uv >= blk * BW) & (uv < (blk + 1) * BW)
                     & (iota < nloc - j * LANES))
                plsc.store_compressed(tmpu.at[pl.ds(0, LANES)], uv, mask=m)
                plsc.store_compressed(tmpb.at[pl.ds(0, LANES)], bv, mask=m)
                n = plsc.all_reduce_population_count(m)[0]

                def el(e, cnt2):
                    u = tmpu[pl.ds(e, LANES)][0]
                    b = tmpb[pl.ds(e, LANES)][0]
                    c = jnp.broadcast_to(u - blk * BW, (LANES,))
                    slot = cnt2 & (RING - 1)
                    for t in range(F // LANES):
                        v = plsc.load_gather(bufs.at[s],
                                             [iota + t * LANES, c])
                        rowring[slot, pl.ds(t * LANES, LANES)] = v
                    pltpu.async_copy(rowring.at[slot],
                                     out_hbm.at[pl.ds(b * F, F)], semrow)

                    @pl.when(cnt2 >= RING)
                    def _():
                        pltpu.make_async_copy(tail_hbm.at[pl.ds(0, F)],
                                              rowring.at[0], semrow).wait()
                    return cnt2 + 1

                return lax.fori_loop(0, n, el, cnt)

            cnt_end = lax.fori_loop(0, nj, jloop, 0)

            def dr(_, __):
                pltpu.make_async_copy(tail_hbm.at[pl.ds(0, F)],
                                      rowring.at[0], semrow).wait()
                return 0
            lax.fori_loop(0, jnp.minimum(cnt_end, RING), dr, 0)

        def body(it, _):
            for s in range(DEPTH):
                kb = it * DEPTH + s

                @pl.when(blk_valid(kb))
                def _(kb=kb, s=s):
                    drain_block(s)
                    extract(kb, s)

                    @pl.when(blk_valid(kb + DEPTH))
                    def _():
                        issue(kb + DEPTH, s)
            return 0

        lax.fori_loop(0, (nb_iters + DEPTH - 1) // DEPTH, body, 0)

        @pl.when(wid == tail_wid)
        def _():
            def tj(j, _):
                uv = locu[pl.ds(j * LANES, LANES)]
                bv = locb[pl.ds(j * LANES, LANES)]
                m = (uv >= tail0) & (iota < nloc - j * LANES)
                plsc.store_compressed(tmpu.at[pl.ds(0, LANES)], uv, mask=m)
                plsc.store_compressed(tmpb.at[pl.ds(0, LANES)], bv, mask=m)
                n = plsc.all_reduce_population_count(m)[0]

                def tel(e, __):
                    u = tmpu[pl.ds(e, LANES)][0]
                    b = tmpb[pl.ds(e, LANES)][0]
                    pltpu.sync_copy(tail_hbm.at[pl.ds((u - tail0) * F, F)],
                                    rowring.at[0])
                    pltpu.sync_copy(rowring.at[0],
                                    out_hbm.at[pl.ds(b * F, F)])
                    return 0
                lax.fori_loop(0, n, tel, 0)
                return 0
            lax.fori_loop(0, nj, tj, 0)

    xlo = wid * X_RANGE
    xhi = jnp.where(wid == NW - 1, N_USERS, xlo + X_RANGE)
    phase(users_hbm, lambda uv: (uv >= xlo) & (uv < xhi),
          XB_PER, lambda kb: wid * XB_PER + kb,
          lambda kb: kb < XB_PER,
          xt_hbm, ue_hbm, X_TAIL0, xtail_hbm, NW - 1)

    def ysel(uv):
        m = ((uv // BW) % NW == wid) & (uv < Y_TAIL0)
        return m | ((uv >= Y_TAIL0) & (wid == 3))

    phase(items_hbm, ysel,
          YB_ITERS, lambda kb: kb * NW + wid,
          lambda kb: (kb < YB_ITERS) & (kb * NW + wid < YB_TOT),
          yt_hbm, ie_hbm, Y_TAIL0, ytail_hbm, 3)


def _dot_body(ue_hbm, ie_hbm, out_hbm, uebuf, iebuf, outv):
    wid = lax.axis_index("s") * NUM_CORES + lax.axis_index("c")
    n_el = outv.shape[0]
    base = wid * n_el
    iota = _iota()
    rot_reduce = _rot_factory(iota)
    pltpu.sync_copy(ue_hbm.at[pl.ds(base * F, n_el * F)], uebuf)
    pltpu.sync_copy(ie_hbm.at[pl.ds(base * F, n_el * F)], iebuf)

    def group(g, _):
        acc = jnp.zeros((LANES,), jnp.float32)
        for l in range(LANES):
            r = g * LANES + l
            p = None
            for t in range(F // LANES):
                q = (uebuf[pl.ds(r * F + t * LANES, LANES)]
                     * iebuf[pl.ds(r * F + t * LANES, LANES)])
                p = q if p is None else p + q
            p = rot_reduce(p)
            acc = jnp.where(iota == l, p, acc)
        outv[pl.ds(g * LANES, LANES)] = acc
        return 0

    lax.fori_loop(0, n_el // LANES, group, 0)
    pltpu.sync_copy(outv, out_hbm.at[pl.ds(base, n_el)])


def _mesh():
    return plsc.VectorSubcoreMesh(core_axis_name="c", subcore_axis_name="s",
                                  num_cores=NUM_CORES,
                                  num_subcores=NUM_SUBCORES)


@functools.cache
def _build_gather():
    return pl.kernel(
        _gather_body,
        out_type=(jax.ShapeDtypeStruct((BATCH * F,), jnp.float32),
                  jax.ShapeDtypeStruct((BATCH * F,), jnp.float32)),
        mesh=_mesh(),
        scratch_types=[
            pltpu.VMEM((DEPTH, F, BW), jnp.float32),
            pltpu.VMEM((BATCH,), jnp.int32),
            pltpu.VMEM((LOC + LANES,), jnp.int32),
            pltpu.VMEM((LOC + LANES,), jnp.int32),
            pltpu.VMEM((2 * LANES,), jnp.int32),
            pltpu.VMEM((2 * LANES,), jnp.int32),
            pltpu.VMEM((RING, F), jnp.float32),
            pltpu.SemaphoreType.DMA,
            pltpu.SemaphoreType.DMA,
            pltpu.SemaphoreType.DMA,
            pltpu.SemaphoreType.DMA,
        ],
        compiler_params=pltpu.CompilerParams(use_tc_tiling_on_sc=True,
                                             needs_layout_passes=False),
    )


@functools.cache
def _build_dot():
    n_el = BATCH // NW
    return pl.kernel(
        _dot_body,
        out_type=jax.ShapeDtypeStruct((BATCH,), jnp.float32),
        mesh=_mesh(),
        scratch_types=[
            pltpu.VMEM((n_el * F,), jnp.float32),
            pltpu.VMEM((n_el * F,), jnp.float32),
            pltpu.VMEM((n_el,), jnp.float32),
        ],
        compiler_params=pltpu.CompilerParams(use_tc_tiling_on_sc=True,
                                             needs_layout_passes=False),
    )


def kernel(users, items, x, y):
    users = users.astype(jnp.int32)
    items = items.astype(jnp.int32)
    xtail = lax.slice(x, (X_TAIL0, 0), (N_USERS, F)).reshape(-1)
    ytail = lax.slice(y, (Y_TAIL0, 0), (N_ITEMS, F)).reshape(-1)
    ue, ie = _build_gather()(users, items, x.T, y.T, xtail, ytail)
    return _build_dot()(ue, ie)
